# baseline (speedup 1.0000x reference)
import functools

import jax
import jax.numpy as jnp
from jax import lax
from jax.experimental import pallas as pl
from jax.experimental.pallas import tpu as pltpu
from jax.experimental.pallas import tpu_sc as plsc

_D = 64
_DPAD = 128
_BATCH = 4096
_SEQ = 200

_info = plsc.get_sparse_core_info()
_NC, _NS, _L = _info.num_cores, _info.num_subcores, _info.num_lanes
_NW = _NC * _NS
_BBLK = _BATCH // _NW
_NBUF = 4
_DI = 4
_DG = 2


def _embed_body(idx_hbm, table_hbm, out_hbm, idx_v, rows_v, comp_v,
                isem, gsem, osem):
    wid = lax.axis_index("s") * _NC + lax.axis_index("c")
    b0 = wid * _BBLK

    def idx_copy(s, slot):
        return pltpu.make_async_copy(idx_hbm.at[s, pl.ds(b0, _BBLK)],
                                     idx_v.at[slot], isem.at[slot])

    def gather(slot):
        return pltpu.make_async_copy(table_hbm.at[idx_v.at[slot]],
                                     rows_v.at[slot], gsem.at[slot])

    def out_copy(s, slot):
        return pltpu.make_async_copy(
            comp_v.at[slot], out_hbm.at[s, :, pl.ds(b0, _BBLK)],
            osem.at[slot])

    bases = [lax.iota(jnp.int32, _L) + (_L * k) for k in range(_BBLK // _L)]

    def transpose_chunk(slot):
        rows = rows_v.at[slot]

        def dstep(d, carry):
            col = jnp.full((_L,), d, jnp.int32)
            for k in range(_BBLK // _L):
                v = plsc.load_gather(rows, [bases[k], col])
                comp_v[slot, d, pl.ds(_L * k, _L)] = v
            return carry

        lax.fori_loop(0, _D, dstep, 0)

    for s in range(_DI):
        idx_copy(s, s % _NBUF).start()
    for s in range(_DG):
        idx_copy(s, s % _NBUF).wait()
        gather(s % _NBUF).start()

    def step(i, carry):
        s0 = i * _NBUF
        for j in range(_NBUF):
            s = s0 + j
            gather(j).wait()

            @pl.when(s >= _NBUF)
            def _():
                out_copy(0, j).wait()

            transpose_chunk(j)
            out_copy(s, j).start()
            si = s + _DI

            @pl.when(si < _SEQ)
            def _():
                idx_copy(si, j).start()

            sg = s + _DG
            gslot = (j + _DG) % _NBUF

            @pl.when(sg < _SEQ)
            def _():
                idx_copy(0, gslot).wait()
                gather(gslot).start()

        return carry

    lax.fori_loop(0, _SEQ // _NBUF, step, 0)

    for j in range(_NBUF):
        out_copy(0, j).wait()


_mesh = plsc.VectorSubcoreMesh(core_axis_name="c", subcore_axis_name="s")

_embed = functools.partial(
    pl.kernel,
    mesh=_mesh,
    out_type=jax.ShapeDtypeStruct((_SEQ, _D, _BATCH), jnp.float32),
    scratch_types=[
        pltpu.VMEM((_NBUF, _BBLK), jnp.int32),
        pltpu.VMEM((_NBUF, _BBLK, _DPAD), jnp.float32),
        pltpu.VMEM((_NBUF, _D, _BBLK), jnp.float32),
        pltpu.SemaphoreType.DMA((_NBUF,)),
        pltpu.SemaphoreType.DMA((_NBUF,)),
        pltpu.SemaphoreType.DMA((_NBUF,)),
    ],
    compiler_params=pltpu.CompilerParams(use_tc_tiling_on_sc=True,
                                         needs_layout_passes=False),
)(_embed_body)


@jax.jit
def kernel(actions, table):
    idx_t = actions.T.astype(jnp.int32)
    tab = jnp.pad(table, ((0, 0), (0, _DPAD - _D)))
    out_t = _embed(idx_t, tab)
    return jnp.transpose(out_t, (2, 0, 1))
